# Initial kernel scaffold; baseline (speedup 1.0000x reference)
import jax
import jax.numpy as jnp
from jax.experimental import pallas as pl


def kernel(feat_idx, w_weight, r_weight):
    raise NotImplementedError("write your pallas kernel here")


# speedup vs baseline: 46.9378x; 46.9378x over previous
import functools

import jax
import jax.numpy as jnp
from jax import lax
from jax.experimental import pallas as pl
from jax.experimental.pallas import tpu as pltpu
from jax.experimental.pallas import tpu_sc as plsc

_NC = 2
_NS = 16
_NW = _NC * _NS
_LANES = 16

_B = 16384
_L = 200
_CHUNK = 16
_ROWS_PER_W = _B // _NW
_NCHUNKS = _ROWS_PER_W // _CHUNK
_PAIRS = _NCHUNKS // 2
_IDX_N = _CHUNK * _L

_W_ADJ = 0.4
_INV_R_ADJ = 0.1

_mesh = plsc.VectorSubcoreMesh(
    core_axis_name="c", subcore_axis_name="s", num_cores=_NC, num_subcores=_NS
)


@functools.partial(
    pl.kernel,
    out_type=jax.ShapeDtypeStruct((_B * 2,), jnp.float32),
    mesh=_mesh,
    scratch_types=[
        pltpu.VMEM((_IDX_N,), jnp.int32),
        pltpu.VMEM((_IDX_N,), jnp.int32),
        pltpu.VMEM((_IDX_N, 4), jnp.float32),
        pltpu.VMEM((_IDX_N, 4), jnp.float32),
        pltpu.VMEM((_ROWS_PER_W * 2,), jnp.float32),
        pltpu.SemaphoreType.DMA,
        pltpu.SemaphoreType.DMA,
        pltpu.SemaphoreType.DMA,
        pltpu.SemaphoreType.DMA,
    ],
    compiler_params=pltpu.CompilerParams(needs_layout_passes=False,
                                         use_tc_tiling_on_sc=False),
)
def _dotprod_nb_sc(idx_hbm, tab_hbm, out_hbm,
                   idx0, idx1, rows0, rows1, res_v,
                   sem_i0, sem_i1, sem_r0, sem_r1):
    wid = lax.axis_index("s") * _NC + lax.axis_index("c")
    row_base = wid * _ROWS_PER_W

    idx_bufs = (idx0, idx1)
    rows_bufs = (rows0, rows1)
    sem_idx = (sem_i0, sem_i1)
    sem_rows = (sem_r0, sem_r1)

    lane = lax.iota(jnp.int32, _LANES)
    lane_l = lane * _L
    col0 = jnp.zeros((_LANES,), jnp.int32)
    col1 = jnp.ones((_LANES,), jnp.int32)
    col2 = jnp.full((_LANES,), 2, jnp.int32)
    zero_acc = jnp.zeros((_LANES,), jnp.float32)

    def idx_src(g):
        start = (row_base + g * _CHUNK) * _L
        return idx_hbm.at[pl.ds(start, _IDX_N)]

    def start_idx(g, b):
        pltpu.async_copy(idx_src(g), idx_bufs[b], sem_idx[b])

    def wait_idx(g, b):
        pltpu.make_async_copy(idx_src(g), idx_bufs[b], sem_idx[b]).wait()

    def start_gather(b):
        pltpu.async_copy(tab_hbm.at[idx_bufs[b]], rows_bufs[b], sem_rows[b])

    def wait_gather(b):
        pltpu.make_async_copy(tab_hbm.at[idx_bufs[b]], rows_bufs[b],
                              sem_rows[b]).wait()

    def process(g, b):
        rows = rows_bufs[b]

        def lbody(l, carry):
            a0, a1 = carry
            rid = lane_l + l
            wv = plsc.load_gather(rows, [rid, col0])
            r0 = plsc.load_gather(rows, [rid, col1])
            r1 = plsc.load_gather(rows, [rid, col2])
            wv = wv + _W_ADJ
            return (a0 + wv * r0, a1 + wv * r1)

        a0, a1 = lax.fori_loop(0, _L, lbody, (zero_acc, zero_acc))
        oid = (g * _CHUNK + lane) * 2
        plsc.store_scatter(res_v, [oid], a0 * _INV_R_ADJ)
        plsc.store_scatter(res_v, [oid + 1], a1 * _INV_R_ADJ)

    start_idx(0, 0)
    wait_idx(0, 0)
    start_gather(0)
    start_idx(1, 1)

    @pl.loop(0, _PAIRS)
    def _pair(p):
        g0 = 2 * p
        wait_gather(0)
        wait_idx(g0 + 1, 1)
        start_gather(1)

        @pl.when(p < _PAIRS - 1)
        def _():
            start_idx(g0 + 2, 0)

        process(g0, 0)
        wait_gather(1)

        @pl.when(p < _PAIRS - 1)
        def _():
            wait_idx(g0 + 2, 0)
            start_gather(0)
            start_idx(g0 + 3, 1)

        process(g0 + 1, 1)

    pltpu.sync_copy(res_v, out_hbm.at[pl.ds(row_base * 2, _ROWS_PER_W * 2)])


def kernel(feat_idx, w_weight, r_weight):
    tab = jnp.concatenate(
        [w_weight, r_weight, jnp.zeros_like(w_weight)], axis=1)
    idx_flat = feat_idx.reshape(-1)
    return _dotprod_nb_sc(idx_flat, tab).reshape(_B, 2)


# speedup vs baseline: 208.6822x; 4.4459x over previous
import functools

import jax
import jax.numpy as jnp
from jax import lax
from jax.experimental import pallas as pl
from jax.experimental.pallas import tpu as pltpu
from jax.experimental.pallas import tpu_sc as plsc

_NC = 2
_NS = 16
_NW = _NC * _NS
_LANES = 16

_B = 16384
_L = 200
_V1 = 1000001
_V_PAD = 1 << 20
_CHUNK = 16
_ROWS_PER_W = _B // _NW
_NCHUNKS = _ROWS_PER_W // _CHUNK
_PAIRS = _NCHUNKS // 2
_IDX_N = _CHUNK * _L

_TROWS = _V_PAD // _NS
_S = 4096
_NSUB = _TROWS // _S

_W_ADJ = 0.4
_INV_R_ADJ = 0.1

_mesh = plsc.VectorSubcoreMesh(
    core_axis_name="c", subcore_axis_name="s", num_cores=_NC, num_subcores=_NS
)


@functools.partial(
    pl.kernel,
    out_type=jax.ShapeDtypeStruct((_B * 2,), jnp.float32),
    mesh=_mesh,
    scratch_types=[
        pltpu.HBM((_NC, _V_PAD, 4), jnp.float32),
        pltpu.VMEM((_S,), jnp.float32),
        pltpu.VMEM((_S,), jnp.float32),
        pltpu.VMEM((_S,), jnp.float32),
        pltpu.VMEM((_S, 4), jnp.float32),
        pltpu.VMEM((_IDX_N,), jnp.int32),
        pltpu.VMEM((_IDX_N,), jnp.int32),
        pltpu.VMEM((_IDX_N, 4), jnp.float32),
        pltpu.VMEM((_IDX_N, 4), jnp.float32),
        pltpu.VMEM((_ROWS_PER_W * 2,), jnp.float32),
        pltpu.SemaphoreType.DMA,
        pltpu.SemaphoreType.DMA,
        pltpu.SemaphoreType.DMA,
        pltpu.SemaphoreType.DMA,
    ],
    compiler_params=pltpu.CompilerParams(needs_layout_passes=False,
                                         use_tc_tiling_on_sc=False),
)
def _dotprod_nb_sc(idx_hbm, w_hbm, r0_hbm, r1_hbm, out_hbm,
                   fused, wbuf, r0buf, r1buf, fbuf,
                   idx0, idx1, rows0, rows1, res_v,
                   sem_i0, sem_i1, sem_r0, sem_r1):
    cid = lax.axis_index("c")
    sid = lax.axis_index("s")
    wid = sid * _NC + cid
    row_base = wid * _ROWS_PER_W

    idx_bufs = (idx0, idx1)
    rows_bufs = (rows0, rows1)
    sem_idx = (sem_i0, sem_i1)
    sem_rows = (sem_r0, sem_r1)

    lane = lax.iota(jnp.int32, _LANES)
    lane_l = lane * _L
    col0 = jnp.zeros((_LANES,), jnp.int32)
    col1 = jnp.ones((_LANES,), jnp.int32)
    col2 = jnp.full((_LANES,), 2, jnp.int32)
    zero_acc = jnp.zeros((_LANES,), jnp.float32)

    tile_row0 = sid * _TROWS

    @pl.loop(0, _NSUB)
    def _fmt(s):
        base = tile_row0 + s * _S
        pltpu.sync_copy(w_hbm.at[pl.ds(base, _S)], wbuf)
        pltpu.sync_copy(r0_hbm.at[pl.ds(base, _S)], r0buf)
        pltpu.sync_copy(r1_hbm.at[pl.ds(base, _S)], r1buf)

        def gbody(g, carry):
            rid = g * _LANES + lane
            off = pl.ds(g * _LANES, _LANES)
            plsc.store_scatter(fbuf, [rid, col0], wbuf[off])
            plsc.store_scatter(fbuf, [rid, col1], r0buf[off])
            plsc.store_scatter(fbuf, [rid, col2], r1buf[off])
            return carry

        lax.fori_loop(0, _S // _LANES, gbody, 0)
        pltpu.sync_copy(fbuf, fused.at[cid, pl.ds(base, _S), :])

    plsc.subcore_barrier()

    tab = fused.at[cid]

    def idx_src(g):
        start = (row_base + g * _CHUNK) * _L
        return idx_hbm.at[pl.ds(start, _IDX_N)]

    def start_idx(g, b):
        pltpu.async_copy(idx_src(g), idx_bufs[b], sem_idx[b])

    def wait_idx(g, b):
        pltpu.make_async_copy(idx_src(g), idx_bufs[b], sem_idx[b]).wait()

    def start_gather(b):
        pltpu.async_copy(tab.at[idx_bufs[b]], rows_bufs[b], sem_rows[b])

    def wait_gather(b):
        pltpu.make_async_copy(tab.at[idx_bufs[b]], rows_bufs[b],
                              sem_rows[b]).wait()

    def process(g, b):
        rows = rows_bufs[b]

        def lbody(l, carry):
            a0, a1 = carry
            rid = lane_l + l
            wv = plsc.load_gather(rows, [rid, col0])
            r0 = plsc.load_gather(rows, [rid, col1])
            r1 = plsc.load_gather(rows, [rid, col2])
            wv = wv + _W_ADJ
            return (a0 + wv * r0, a1 + wv * r1)

        a0, a1 = lax.fori_loop(0, _L, lbody, (zero_acc, zero_acc))
        oid = (g * _CHUNK + lane) * 2
        plsc.store_scatter(res_v, [oid], a0 * _INV_R_ADJ)
        plsc.store_scatter(res_v, [oid + 1], a1 * _INV_R_ADJ)

    start_idx(0, 0)
    wait_idx(0, 0)
    start_gather(0)
    start_idx(1, 1)

    @pl.loop(0, _PAIRS)
    def _pair(p):
        g0 = 2 * p
        wait_gather(0)
        wait_idx(g0 + 1, 1)
        start_gather(1)

        @pl.when(p < _PAIRS - 1)
        def _():
            start_idx(g0 + 2, 0)

        process(g0, 0)
        wait_gather(1)

        @pl.when(p < _PAIRS - 1)
        def _():
            wait_idx(g0 + 2, 0)
            start_gather(0)
            start_idx(g0 + 3, 1)

        process(g0 + 1, 1)

    pltpu.sync_copy(res_v, out_hbm.at[pl.ds(row_base * 2, _ROWS_PER_W * 2)])


def kernel(feat_idx, w_weight, r_weight):
    pad = (0, _V_PAD - _V1)
    w_flat = jnp.pad(w_weight.reshape(-1), pad)
    r0 = jnp.pad(r_weight[:, 0], pad)
    r1 = jnp.pad(r_weight[:, 1], pad)
    idx_flat = feat_idx.reshape(-1)
    return _dotprod_nb_sc(idx_flat, w_flat, r0, r1).reshape(_B, 2)
